# Initial kernel scaffold; baseline (speedup 1.0000x reference)
import jax
import jax.numpy as jnp
from jax.experimental import pallas as pl


def kernel(x, edge_index, edge_attr, batch, W1, b1, gamma, beta, We, be, Wc0, bc0, Wc1, bc1, Wc2, bc2, Wm1, bm1, Wm2, bm2):
    raise NotImplementedError("write your pallas kernel here")


# speedup vs baseline: 6.6596x; 6.6596x over previous
import functools

import jax
import jax.numpy as jnp
from jax import lax
from jax.experimental import pallas as pl
from jax.experimental.pallas import tpu as pltpu
from jax.experimental.pallas import tpu_sc as plsc

N_PAD = 10240
ROWS = 1280
GRID = 8
H = 128
NG = 64
NC, NS = 2, 16
NW = NC * NS
BLK = 128
NBLK = 80
E_PAD = NW * NBLK * BLK
RPT = N_PAD // NS

_SC_MESH = plsc.VectorSubcoreMesh(
    core_axis_name="c", subcore_axis_name="s", num_cores=NC, num_subcores=NS)


@functools.partial(
    pl.kernel,
    out_type=jax.ShapeDtypeStruct((NC * N_PAD, H), jnp.float32),
    mesh=_SC_MESH,
    scratch_types=[
        pltpu.VMEM((NBLK // 2, BLK), jnp.int32),
        pltpu.VMEM((BLK, H), jnp.float32),
        pltpu.VMEM_SHARED((N_PAD, H), jnp.float32),
    ],
)
def _sc_degree(col_hbm, ones_hbm, zeros_hbm, out_hbm, colv, onesv, acc):
    cid = lax.axis_index("c")
    sid = lax.axis_index("s")
    wid = cid * NS + sid
    pltpu.sync_copy(ones_hbm, onesv)
    pltpu.sync_copy(zeros_hbm.at[pl.ds(sid * RPT, RPT)],
                    acc.at[pl.ds(sid * RPT, RPT)])
    plsc.subcore_barrier()

    def body(j, carry):
        pltpu.sync_copy(onesv, acc.at[colv.at[j]], add=True)
        return carry

    for ph in range(2):
        pltpu.sync_copy(col_hbm.at[2 * wid + ph], colv)
        lax.fori_loop(0, NBLK // 2, body, 0)
    plsc.subcore_barrier()
    pltpu.sync_copy(acc.at[pl.ds(sid * RPT, RPT)],
                    out_hbm.at[pl.ds(cid * N_PAD + sid * RPT, RPT)])


HALF = NBLK // 2


@functools.partial(
    pl.kernel,
    out_type=jax.ShapeDtypeStruct((NC * N_PAD, H), jnp.float32),
    mesh=_SC_MESH,
    scratch_types=[
        pltpu.VMEM((HALF, BLK), jnp.int32),
        pltpu.VMEM((HALF, BLK), jnp.int32),
        pltpu.VMEM((BLK, H), jnp.float32),
        pltpu.VMEM((BLK, H), jnp.float32),
        pltpu.VMEM_SHARED((N_PAD, H), jnp.float32),
        pltpu.SemaphoreType.DMA,
        pltpu.SemaphoreType.DMA,
    ],
)
def _sc_agg(rowi_hbm, coli_hbm, hp_hbm, zeros_hbm, out_hbm,
            rowv, colv, buf0, buf1, acc, sem0, sem1):
    cid = lax.axis_index("c")
    sid = lax.axis_index("s")
    wid = cid * NS + sid
    pltpu.sync_copy(zeros_hbm.at[pl.ds(sid * RPT, RPT)],
                    acc.at[pl.ds(sid * RPT, RPT)])
    plsc.subcore_barrier()

    bufs = (buf0, buf1)
    sems = (sem0, sem1)
    for ph in range(2):
        pltpu.sync_copy(rowi_hbm.at[2 * wid + ph], rowv)
        pltpu.sync_copy(coli_hbm.at[2 * wid + ph], colv)
        pltpu.async_copy(hp_hbm.at[rowv.at[0]], buf0, sem0)
        pltpu.async_copy(hp_hbm.at[rowv.at[1]], buf1, sem1)

        def body(t, carry):
            for b in range(2):
                j = 2 * t + b
                pltpu.make_async_copy(
                    hp_hbm.at[rowv.at[j]], bufs[b], sems[b]).wait()
                pltpu.sync_copy(bufs[b], acc.at[colv.at[j]], add=True)

                @pl.when(j + 2 < HALF)
                def _():
                    pltpu.async_copy(hp_hbm.at[rowv.at[j + 2]], bufs[b], sems[b])
            return carry

        lax.fori_loop(0, HALF // 2, body, 0)
    plsc.subcore_barrier()
    pltpu.sync_copy(acc.at[pl.ds(sid * RPT, RPT)],
                    out_hbm.at[pl.ds(cid * N_PAD + sid * RPT, RPT)])


def _enc_body(x_ref, w1_ref, b1_ref, g_ref, be_ref, wc_ref, o_ref):
    h = jnp.maximum(
        jnp.dot(x_ref[...], w1_ref[...], preferred_element_type=jnp.float32)
        + b1_ref[...], 0.0)
    mu = jnp.mean(h, axis=-1, keepdims=True)
    var = jnp.mean((h - mu) * (h - mu), axis=-1, keepdims=True)
    h = (h - mu) * lax.rsqrt(var + 1e-5) * g_ref[...] + be_ref[...]
    o_ref[...] = jnp.dot(h, wc_ref[...], preferred_element_type=jnp.float32)


def _dinv(cnt_ref):
    return lax.rsqrt(cnt_ref[0, :, 0:1] + cnt_ref[1, :, 0:1] + 1.0)


def _scale_body(cnt_ref, hw_ref, o_ref):
    o_ref[...] = hw_ref[...] * _dinv(cnt_ref)


def _combine_body(cnt_ref, p_ref, hp_ref, b_ref, w_ref, o_ref):
    dinv = _dinv(cnt_ref)
    h = jnp.maximum((p_ref[0] + p_ref[1] + hp_ref[...]) * dinv + b_ref[...], 0.0)
    o_ref[...] = jnp.dot(h, w_ref[...], preferred_element_type=jnp.float32) * dinv


def _final_body(cnt_ref, p_ref, hp_ref, b_ref, batch_ref,
                wm1_ref, bm1_ref, wm2_ref, bm2_ref, o_ref, sums, cnts):
    i = pl.program_id(0)

    @pl.when(i == 0)
    def _():
        sums[...] = jnp.zeros_like(sums)
        cnts[...] = jnp.zeros_like(cnts)

    dinv = _dinv(cnt_ref)
    h = jnp.maximum((p_ref[0] + p_ref[1] + hp_ref[...]) * dinv + b_ref[...], 0.0)
    b = batch_ref[0, 0, :]
    onehot = (b[:, None] == lax.broadcasted_iota(jnp.int32, (ROWS, NG), 1)
              ).astype(jnp.float32)
    sums[...] += lax.dot_general(onehot, h, (((0,), (0,)), ((), ())),
                                 preferred_element_type=jnp.float32)
    cnts[...] += jnp.sum(onehot, axis=0, keepdims=True)

    @pl.when(i == pl.num_programs(0) - 1)
    def _():
        means = sums[...] / jnp.maximum(cnts[...], 1.0).T
        z = jnp.maximum(
            jnp.dot(means, wm1_ref[...], preferred_element_type=jnp.float32)
            + bm1_ref[...], 0.0)
        o_ref[...] = (jnp.dot(z, wm2_ref[...], preferred_element_type=jnp.float32)
                      + bm2_ref[...])


def _full(shape):
    nd = len(shape)
    return pl.BlockSpec(shape, lambda i: (0,) * nd)


_ROWBLOCK = pl.BlockSpec((ROWS, H), lambda i: (i, 0))
_PBLOCK = pl.BlockSpec((NC, ROWS, H), lambda i: (0, i, 0))
_CNTBLOCK = _PBLOCK

_tc_encoder = pl.pallas_call(
    _enc_body, grid=(GRID,),
    in_specs=[_ROWBLOCK, _full((H, H)), _full((1, H)), _full((1, H)),
              _full((1, H)), _full((H, H))],
    out_specs=_ROWBLOCK,
    out_shape=jax.ShapeDtypeStruct((N_PAD, H), jnp.float32))

_tc_scale = pl.pallas_call(
    _scale_body, grid=(GRID,),
    in_specs=[_CNTBLOCK, _ROWBLOCK],
    out_specs=_ROWBLOCK,
    out_shape=jax.ShapeDtypeStruct((N_PAD, H), jnp.float32))

_tc_combine = pl.pallas_call(
    _combine_body, grid=(GRID,),
    in_specs=[_CNTBLOCK, _PBLOCK, _ROWBLOCK, _full((1, H)), _full((H, H))],
    out_specs=_ROWBLOCK,
    out_shape=jax.ShapeDtypeStruct((N_PAD, H), jnp.float32))

_tc_final = pl.pallas_call(
    _final_body, grid=(GRID,),
    in_specs=[_CNTBLOCK, _PBLOCK, _ROWBLOCK, _full((1, H)),
              pl.BlockSpec((1, 1, ROWS), lambda i: (i, 0, 0)),
              _full((H, NG)), _full((1, NG)), _full((NG, 1)), _full((1, 1))],
    out_specs=_full((NG, 1)),
    out_shape=jax.ShapeDtypeStruct((NG, 1), jnp.float32),
    scratch_shapes=[pltpu.VMEM((NG, H), jnp.float32),
                    pltpu.VMEM((1, NG), jnp.float32)])


def kernel(x, edge_index, edge_attr, batch, W1, b1, gamma, beta, We, be,
           Wc0, bc0, Wc1, bc1, Wc2, bc2, Wm1, bm1, Wm2, bm2):
    n = x.shape[0]
    f32 = jnp.float32
    i32 = jnp.int32

    x_pad = jnp.zeros((N_PAD, H), f32).at[:n].set(x)
    epad = E_PAD - edge_index.shape[1]
    row3 = jnp.concatenate(
        [edge_index[0].astype(i32), jnp.full((epad,), n, i32)]
    ).reshape(NW * 2, HALF, BLK)
    col3 = jnp.concatenate(
        [edge_index[1].astype(i32), jnp.full((epad,), n, i32)]
    ).reshape(NW * 2, HALF, BLK)
    batch3 = jnp.concatenate(
        [batch.astype(i32), jnp.full((N_PAD - n,), NG, i32)]
    ).reshape(GRID, 1, ROWS)
    zeros_big = jnp.zeros((N_PAD, H), f32)
    ones_blk = jnp.ones((BLK, H), f32)
    b1r, gr, ber = b1.reshape(1, H), gamma.reshape(1, H), beta.reshape(1, H)
    bc0r, bc1r, bc2r = bc0.reshape(1, H), bc1.reshape(1, H), bc2.reshape(1, H)
    bm1r, bm2r = bm1.reshape(1, NG), bm2.reshape(1, 1)

    cnt = _sc_degree(col3, ones_blk, zeros_big).reshape(NC, N_PAD, H)
    hw0 = _tc_encoder(x_pad, W1, b1r, gr, ber, Wc0)
    hp0 = _tc_scale(cnt, hw0)
    p1 = _sc_agg(row3, col3, hp0, zeros_big).reshape(NC, N_PAD, H)
    hp1 = _tc_combine(cnt, p1, hp0, bc0r, Wc1)
    p2 = _sc_agg(row3, col3, hp1, zeros_big).reshape(NC, N_PAD, H)
    hp2 = _tc_combine(cnt, p2, hp1, bc1r, Wc2)
    p3 = _sc_agg(row3, col3, hp2, zeros_big).reshape(NC, N_PAD, H)
    return _tc_final(cnt, p3, hp2, bc2r, batch3, Wm1, bm1r, Wm2, bm2r)


# speedup vs baseline: 24.5451x; 3.6857x over previous
import functools

import jax
import jax.numpy as jnp
from jax import lax
from jax.experimental import pallas as pl
from jax.experimental.pallas import tpu as pltpu
from jax.experimental.pallas import tpu_sc as plsc

N_PAD = 10240
ROWS = 1280
GRID = 8
H = 128
NG = 64
NC, NS = 2, 16
NW = NC * NS
BLK = 128
NBLK = 80
E_PAD = NW * NBLK * BLK
RPT = N_PAD // NS

_SC_MESH = plsc.VectorSubcoreMesh(
    core_axis_name="c", subcore_axis_name="s", num_cores=NC, num_subcores=NS)


@functools.partial(
    pl.kernel,
    out_type=jax.ShapeDtypeStruct((NC * N_PAD, H), jnp.float32),
    mesh=_SC_MESH,
    scratch_types=[
        pltpu.VMEM((NBLK // 2, BLK), jnp.int32),
        pltpu.VMEM((BLK, H), jnp.float32),
        pltpu.VMEM_SHARED((N_PAD, H), jnp.float32),
    ],
)
def _sc_degree(col_hbm, ones_hbm, zeros_hbm, out_hbm, colv, onesv, acc):
    cid = lax.axis_index("c")
    sid = lax.axis_index("s")
    wid = cid * NS + sid
    pltpu.sync_copy(ones_hbm, onesv)
    pltpu.sync_copy(zeros_hbm.at[pl.ds(sid * RPT, RPT)],
                    acc.at[pl.ds(sid * RPT, RPT)])
    plsc.subcore_barrier()

    def body(j, carry):
        pltpu.sync_copy(onesv, acc.at[colv.at[j]], add=True)
        return carry

    for ph in range(2):
        pltpu.sync_copy(col_hbm.at[2 * wid + ph], colv)
        lax.fori_loop(0, NBLK // 2, body, 0)
    plsc.subcore_barrier()
    pltpu.sync_copy(acc.at[pl.ds(sid * RPT, RPT)],
                    out_hbm.at[pl.ds(cid * N_PAD + sid * RPT, RPT)])


HALF = NBLK // 2


@functools.partial(
    pl.kernel,
    out_type=jax.ShapeDtypeStruct((NC * N_PAD, H), jnp.float32),
    mesh=_SC_MESH,
    scratch_types=[
        pltpu.VMEM((HALF, BLK), jnp.int32),
        pltpu.VMEM((HALF, BLK), jnp.int32),
        pltpu.VMEM((BLK, H), jnp.float32),
        pltpu.VMEM((BLK, H), jnp.float32),
        pltpu.VMEM_SHARED((N_PAD, H), jnp.float32),
        pltpu.SemaphoreType.DMA,
        pltpu.SemaphoreType.DMA,
    ],
)
def _sc_agg(rowi_hbm, coli_hbm, hp_hbm, zeros_hbm, out_hbm,
            rowv, colv, buf0, buf1, acc, sem0, sem1):
    cid = lax.axis_index("c")
    sid = lax.axis_index("s")
    wid = cid * NS + sid
    pltpu.sync_copy(zeros_hbm.at[pl.ds(sid * RPT, RPT)],
                    acc.at[pl.ds(sid * RPT, RPT)])
    plsc.subcore_barrier()

    bufs = (buf0, buf1)
    sems = (sem0, sem1)
    for ph in range(2):
        pltpu.sync_copy(rowi_hbm.at[2 * wid + ph], rowv)
        pltpu.sync_copy(coli_hbm.at[2 * wid + ph], colv)
        pltpu.async_copy(hp_hbm.at[rowv.at[0]], buf0, sem0)
        pltpu.async_copy(hp_hbm.at[rowv.at[1]], buf1, sem1)

        def body(t, carry):
            for b in range(2):
                j = 2 * t + b
                pltpu.make_async_copy(
                    hp_hbm.at[rowv.at[j]], bufs[b], sems[b]).wait()
                pltpu.sync_copy(bufs[b], acc.at[colv.at[j]], add=True)

                @pl.when(j + 2 < HALF)
                def _():
                    pltpu.async_copy(hp_hbm.at[rowv.at[j + 2]], bufs[b], sems[b])
            return carry

        lax.fori_loop(0, HALF // 2, body, 0)
    plsc.subcore_barrier()
    pltpu.sync_copy(acc.at[pl.ds(sid * RPT, RPT)],
                    out_hbm.at[pl.ds(cid * N_PAD + sid * RPT, RPT)])


def _enc_body(x_ref, w1_ref, b1_ref, g_ref, be_ref, wc_ref, o_ref):
    h = jnp.maximum(
        jnp.dot(x_ref[...], w1_ref[...], preferred_element_type=jnp.float32)
        + b1_ref[...], 0.0)
    mu = jnp.mean(h, axis=-1, keepdims=True)
    var = jnp.mean((h - mu) * (h - mu), axis=-1, keepdims=True)
    h = (h - mu) * lax.rsqrt(var + 1e-5) * g_ref[...] + be_ref[...]
    o_ref[...] = jnp.dot(h, wc_ref[...], preferred_element_type=jnp.float32)


def _dinv(cnt_ref):
    return lax.rsqrt(cnt_ref[0, :, 0:1] + cnt_ref[1, :, 0:1] + 1.0)


def _scale_body(cnt_ref, hw_ref, o_ref):
    o_ref[...] = hw_ref[...] * _dinv(cnt_ref)


def _combine_body(cnt_ref, p_ref, hp_ref, b_ref, w_ref, o_ref):
    dinv = _dinv(cnt_ref)
    h = jnp.maximum((p_ref[0] + p_ref[1] + hp_ref[...]) * dinv + b_ref[...], 0.0)
    o_ref[...] = jnp.dot(h, w_ref[...], preferred_element_type=jnp.float32) * dinv


def _final_body(cnt_ref, p_ref, hp_ref, b_ref, batch_ref,
                wm1_ref, bm1_ref, wm2_ref, bm2_ref, o_ref, sums, cnts):
    i = pl.program_id(0)

    @pl.when(i == 0)
    def _():
        sums[...] = jnp.zeros_like(sums)
        cnts[...] = jnp.zeros_like(cnts)

    dinv = _dinv(cnt_ref)
    h = jnp.maximum((p_ref[0] + p_ref[1] + hp_ref[...]) * dinv + b_ref[...], 0.0)
    b = batch_ref[0, 0, :]
    onehot = (b[:, None] == lax.broadcasted_iota(jnp.int32, (ROWS, NG), 1)
              ).astype(jnp.float32)
    sums[...] += lax.dot_general(onehot, h, (((0,), (0,)), ((), ())),
                                 preferred_element_type=jnp.float32)
    cnts[...] += jnp.sum(onehot, axis=0, keepdims=True)

    @pl.when(i == pl.num_programs(0) - 1)
    def _():
        means = sums[...] / jnp.maximum(cnts[...], 1.0).T
        z = jnp.maximum(
            jnp.dot(means, wm1_ref[...], preferred_element_type=jnp.float32)
            + bm1_ref[...], 0.0)
        o_ref[...] = (jnp.dot(z, wm2_ref[...], preferred_element_type=jnp.float32)
                      + bm2_ref[...])


def _full(shape):
    nd = len(shape)
    return pl.BlockSpec(shape, lambda i: (0,) * nd)


_ROWBLOCK = pl.BlockSpec((ROWS, H), lambda i: (i, 0))
_PBLOCK = pl.BlockSpec((NC, ROWS, H), lambda i: (0, i, 0))
_CNTBLOCK = _PBLOCK

_tc_encoder = pl.pallas_call(
    _enc_body, grid=(GRID,),
    in_specs=[_ROWBLOCK, _full((H, H)), _full((1, H)), _full((1, H)),
              _full((1, H)), _full((H, H))],
    out_specs=_ROWBLOCK,
    out_shape=jax.ShapeDtypeStruct((N_PAD, H), jnp.float32))

_tc_scale = pl.pallas_call(
    _scale_body, grid=(GRID,),
    in_specs=[_CNTBLOCK, _ROWBLOCK],
    out_specs=_ROWBLOCK,
    out_shape=jax.ShapeDtypeStruct((N_PAD, H), jnp.float32))

_tc_combine = pl.pallas_call(
    _combine_body, grid=(GRID,),
    in_specs=[_CNTBLOCK, _PBLOCK, _ROWBLOCK, _full((1, H)), _full((H, H))],
    out_specs=_ROWBLOCK,
    out_shape=jax.ShapeDtypeStruct((N_PAD, H), jnp.float32))

_tc_final = pl.pallas_call(
    _final_body, grid=(GRID,),
    in_specs=[_CNTBLOCK, _PBLOCK, _ROWBLOCK, _full((1, H)),
              pl.BlockSpec((1, 1, ROWS), lambda i: (i, 0, 0)),
              _full((H, NG)), _full((1, NG)), _full((NG, 1)), _full((1, 1))],
    out_specs=_full((NG, 1)),
    out_shape=jax.ShapeDtypeStruct((NG, 1), jnp.float32),
    scratch_shapes=[pltpu.VMEM((NG, H), jnp.float32),
                    pltpu.VMEM((1, NG), jnp.float32)])


def kernel(x, edge_index, edge_attr, batch, W1, b1, gamma, beta, We, be,
           Wc0, bc0, Wc1, bc1, Wc2, bc2, Wm1, bm1, Wm2, bm2):
    n = x.shape[0]
    f32 = jnp.float32
    i32 = jnp.int32

    x_pad = jnp.zeros((N_PAD, H), f32).at[:n].set(x)
    epad = E_PAD - edge_index.shape[1]
    dummy = n + (jnp.arange(epad, dtype=i32) % (N_PAD - n))
    row3 = jnp.concatenate(
        [edge_index[0].astype(i32), dummy]).reshape(NW * 2, HALF, BLK)
    col3 = jnp.concatenate(
        [edge_index[1].astype(i32), dummy]).reshape(NW * 2, HALF, BLK)
    batch3 = jnp.concatenate(
        [batch.astype(i32), jnp.full((N_PAD - n,), NG, i32)]
    ).reshape(GRID, 1, ROWS)
    zeros_big = jnp.zeros((N_PAD, H), f32)
    ones_blk = jnp.ones((BLK, H), f32)
    b1r, gr, ber = b1.reshape(1, H), gamma.reshape(1, H), beta.reshape(1, H)
    bc0r, bc1r, bc2r = bc0.reshape(1, H), bc1.reshape(1, H), bc2.reshape(1, H)
    bm1r, bm2r = bm1.reshape(1, NG), bm2.reshape(1, 1)

    cnt = _sc_degree(col3, ones_blk, zeros_big).reshape(NC, N_PAD, H)
    hw0 = _tc_encoder(x_pad, W1, b1r, gr, ber, Wc0)
    hp0 = _tc_scale(cnt, hw0)
    p1 = _sc_agg(row3, col3, hp0, zeros_big).reshape(NC, N_PAD, H)
    hp1 = _tc_combine(cnt, p1, hp0, bc0r, Wc1)
    p2 = _sc_agg(row3, col3, hp1, zeros_big).reshape(NC, N_PAD, H)
    hp2 = _tc_combine(cnt, p2, hp1, bc1r, Wc2)
    p3 = _sc_agg(row3, col3, hp2, zeros_big).reshape(NC, N_PAD, H)
    return _tc_final(cnt, p3, hp2, bc2r, batch3, Wm1, bm1r, Wm2, bm2r)


# speedup vs baseline: 25.2227x; 1.0276x over previous
import functools

import jax
import jax.numpy as jnp
from jax import lax
from jax.experimental import pallas as pl
from jax.experimental.pallas import tpu as pltpu
from jax.experimental.pallas import tpu_sc as plsc

N_PAD = 10240
ROWS = 1280
GRID = 8
H = 128
NG = 64
NC, NS = 2, 16
NW = NC * NS
BLK = 128
NBLK = 80
E_PAD = NW * NBLK * BLK
RPT = N_PAD // NS

_SC_MESH = plsc.VectorSubcoreMesh(
    core_axis_name="c", subcore_axis_name="s", num_cores=NC, num_subcores=NS)


def _fill_zero(zbuf):
    zero = jnp.zeros((16,), jnp.float32)

    def body(r, carry):
        for c in range(H // 16):
            zbuf[r, pl.ds(c * 16, 16)] = zero
        return carry

    lax.fori_loop(0, BLK, body, 0)


def _zero_slice(acc, zbuf, sid):
    _fill_zero(zbuf)
    for k in range(RPT // BLK):
        pltpu.sync_copy(zbuf, acc.at[pl.ds(sid * RPT + k * BLK, BLK)])


@functools.partial(
    pl.kernel,
    out_type=jax.ShapeDtypeStruct((NC * N_PAD, H), jnp.float32),
    mesh=_SC_MESH,
    scratch_types=[
        pltpu.VMEM((NBLK // 2, BLK), jnp.int32),
        pltpu.VMEM((BLK, H), jnp.float32),
        pltpu.VMEM((BLK, H), jnp.float32),
        pltpu.VMEM_SHARED((N_PAD, H), jnp.float32),
    ],
)
def _sc_degree(col_hbm, ones_hbm, out_hbm, colv, onesv, zbuf, acc):
    cid = lax.axis_index("c")
    sid = lax.axis_index("s")
    wid = cid * NS + sid
    pltpu.sync_copy(ones_hbm, onesv)
    _zero_slice(acc, zbuf, sid)
    plsc.subcore_barrier()

    def body(j, carry):
        pltpu.sync_copy(onesv, acc.at[colv.at[j]], add=True)
        return carry

    for ph in range(2):
        pltpu.sync_copy(col_hbm.at[2 * wid + ph], colv)
        lax.fori_loop(0, NBLK // 2, body, 0)
    plsc.subcore_barrier()
    pltpu.sync_copy(acc.at[pl.ds(sid * RPT, RPT)],
                    out_hbm.at[pl.ds(cid * N_PAD + sid * RPT, RPT)])


HALF = NBLK // 2


@functools.partial(
    pl.kernel,
    out_type=jax.ShapeDtypeStruct((NC * N_PAD, H), jnp.float32),
    mesh=_SC_MESH,
    scratch_types=[
        pltpu.VMEM((HALF, BLK), jnp.int32),
        pltpu.VMEM((HALF, BLK), jnp.int32),
        pltpu.VMEM((BLK, H), jnp.float32),
        pltpu.VMEM((BLK, H), jnp.float32),
        pltpu.VMEM_SHARED((N_PAD, H), jnp.float32),
        pltpu.SemaphoreType.DMA,
        pltpu.SemaphoreType.DMA,
    ],
)
def _sc_agg(rowi_hbm, coli_hbm, hp_hbm, out_hbm,
            rowv, colv, buf0, buf1, acc, sem0, sem1):
    cid = lax.axis_index("c")
    sid = lax.axis_index("s")
    wid = cid * NS + sid
    _zero_slice(acc, buf0, sid)
    plsc.subcore_barrier()

    bufs = (buf0, buf1)
    sems = (sem0, sem1)
    for ph in range(2):
        pltpu.sync_copy(rowi_hbm.at[2 * wid + ph], rowv)
        pltpu.sync_copy(coli_hbm.at[2 * wid + ph], colv)
        pltpu.async_copy(hp_hbm.at[rowv.at[0]], buf0, sem0)
        pltpu.async_copy(hp_hbm.at[rowv.at[1]], buf1, sem1)

        def body(t, carry):
            for b in range(2):
                j = 2 * t + b
                pltpu.make_async_copy(
                    hp_hbm.at[rowv.at[j]], bufs[b], sems[b]).wait()
                pltpu.sync_copy(bufs[b], acc.at[colv.at[j]], add=True)

                @pl.when(j + 2 < HALF)
                def _():
                    pltpu.async_copy(hp_hbm.at[rowv.at[j + 2]], bufs[b], sems[b])
            return carry

        lax.fori_loop(0, HALF // 2, body, 0)
    plsc.subcore_barrier()
    pltpu.sync_copy(acc.at[pl.ds(sid * RPT, RPT)],
                    out_hbm.at[pl.ds(cid * N_PAD + sid * RPT, RPT)])


def _dinv(cnt_ref):
    return lax.rsqrt(cnt_ref[0, :, 0:1] + cnt_ref[1, :, 0:1] + 1.0)


def _enc_body(cnt_ref, x_ref, w1_ref, b1_ref, g_ref, be_ref, wc_ref, o_ref):
    h = jnp.maximum(
        jnp.dot(x_ref[...], w1_ref[...], preferred_element_type=jnp.float32)
        + b1_ref[...], 0.0)
    mu = jnp.mean(h, axis=-1, keepdims=True)
    var = jnp.mean((h - mu) * (h - mu), axis=-1, keepdims=True)
    h = (h - mu) * lax.rsqrt(var + 1e-5) * g_ref[...] + be_ref[...]
    o_ref[...] = jnp.dot(h, wc_ref[...],
                         preferred_element_type=jnp.float32) * _dinv(cnt_ref)


def _combine_body(cnt_ref, p_ref, hp_ref, b_ref, w_ref, o_ref):
    dinv = _dinv(cnt_ref)
    h = jnp.maximum((p_ref[0] + p_ref[1] + hp_ref[...]) * dinv + b_ref[...], 0.0)
    o_ref[...] = jnp.dot(h, w_ref[...], preferred_element_type=jnp.float32) * dinv


def _final_body(cnt_ref, p_ref, hp_ref, b_ref, batch_ref,
                wm1_ref, bm1_ref, wm2_ref, bm2_ref, o_ref, sums, cnts):
    i = pl.program_id(0)

    @pl.when(i == 0)
    def _():
        sums[...] = jnp.zeros_like(sums)
        cnts[...] = jnp.zeros_like(cnts)

    dinv = _dinv(cnt_ref)
    h = jnp.maximum((p_ref[0] + p_ref[1] + hp_ref[...]) * dinv + b_ref[...], 0.0)
    b = batch_ref[0, 0, :]
    onehot = (b[:, None] == lax.broadcasted_iota(jnp.int32, (ROWS, NG), 1)
              ).astype(jnp.float32)
    sums[...] += lax.dot_general(onehot, h, (((0,), (0,)), ((), ())),
                                 preferred_element_type=jnp.float32)
    cnts[...] += jnp.sum(onehot, axis=0, keepdims=True)

    @pl.when(i == pl.num_programs(0) - 1)
    def _():
        means = sums[...] / jnp.maximum(cnts[...], 1.0).T
        z = jnp.maximum(
            jnp.dot(means, wm1_ref[...], preferred_element_type=jnp.float32)
            + bm1_ref[...], 0.0)
        o_ref[...] = (jnp.dot(z, wm2_ref[...], preferred_element_type=jnp.float32)
                      + bm2_ref[...])


def _full(shape):
    nd = len(shape)
    return pl.BlockSpec(shape, lambda i: (0,) * nd)


_ROWBLOCK = pl.BlockSpec((ROWS, H), lambda i: (i, 0))
_PBLOCK = pl.BlockSpec((NC, ROWS, H), lambda i: (0, i, 0))
_CNTBLOCK = pl.BlockSpec((NC, ROWS, 16), lambda i: (0, i, 0))

_tc_encoder = pl.pallas_call(
    _enc_body, grid=(GRID,),
    in_specs=[_CNTBLOCK, _ROWBLOCK, _full((H, H)), _full((1, H)),
              _full((1, H)), _full((1, H)), _full((H, H))],
    out_specs=_ROWBLOCK,
    out_shape=jax.ShapeDtypeStruct((N_PAD, H), jnp.float32))

_tc_combine = pl.pallas_call(
    _combine_body, grid=(GRID,),
    in_specs=[_CNTBLOCK, _PBLOCK, _ROWBLOCK, _full((1, H)), _full((H, H))],
    out_specs=_ROWBLOCK,
    out_shape=jax.ShapeDtypeStruct((N_PAD, H), jnp.float32))

_tc_final = pl.pallas_call(
    _final_body, grid=(GRID,),
    in_specs=[_CNTBLOCK, _PBLOCK, _ROWBLOCK, _full((1, H)),
              pl.BlockSpec((1, 1, ROWS), lambda i: (i, 0, 0)),
              _full((H, NG)), _full((1, NG)), _full((NG, 1)), _full((1, 1))],
    out_specs=_full((NG, 1)),
    out_shape=jax.ShapeDtypeStruct((NG, 1), jnp.float32),
    scratch_shapes=[pltpu.VMEM((NG, H), jnp.float32),
                    pltpu.VMEM((1, NG), jnp.float32)])


def kernel(x, edge_index, edge_attr, batch, W1, b1, gamma, beta, We, be,
           Wc0, bc0, Wc1, bc1, Wc2, bc2, Wm1, bm1, Wm2, bm2):
    n = x.shape[0]
    f32 = jnp.float32
    i32 = jnp.int32

    x_pad = jnp.zeros((N_PAD, H), f32).at[:n].set(x)
    epad = E_PAD - edge_index.shape[1]
    dummy = n + (jnp.arange(epad, dtype=i32) % (N_PAD - n))
    row3 = jnp.concatenate(
        [edge_index[0].astype(i32), dummy]).reshape(NW * 2, HALF, BLK)
    col3 = jnp.concatenate(
        [edge_index[1].astype(i32), dummy]).reshape(NW * 2, HALF, BLK)
    batch3 = jnp.concatenate(
        [batch.astype(i32), jnp.full((N_PAD - n,), NG, i32)]
    ).reshape(GRID, 1, ROWS)
    ones_blk = jnp.ones((BLK, H), f32)
    b1r, gr, ber = b1.reshape(1, H), gamma.reshape(1, H), beta.reshape(1, H)
    bc0r, bc1r, bc2r = bc0.reshape(1, H), bc1.reshape(1, H), bc2.reshape(1, H)
    bm1r, bm2r = bm1.reshape(1, NG), bm2.reshape(1, 1)

    cnt = _sc_degree(col3, ones_blk).reshape(
        NC, N_PAD, H)[:, :, :16]
    hp0 = _tc_encoder(cnt, x_pad, W1, b1r, gr, ber, Wc0)
    p1 = _sc_agg(row3, col3, hp0).reshape(NC, N_PAD, H)
    hp1 = _tc_combine(cnt, p1, hp0, bc0r, Wc1)
    p2 = _sc_agg(row3, col3, hp1).reshape(NC, N_PAD, H)
    hp2 = _tc_combine(cnt, p2, hp1, bc1r, Wc2)
    p3 = _sc_agg(row3, col3, hp2).reshape(NC, N_PAD, H)
    return _tc_final(cnt, p3, hp2, bc2r, batch3, Wm1, bm1r, Wm2, bm2r)


# speedup vs baseline: 25.4848x; 1.0104x over previous
import functools

import jax
import jax.numpy as jnp
from jax import lax
from jax.experimental import pallas as pl
from jax.experimental.pallas import tpu as pltpu
from jax.experimental.pallas import tpu_sc as plsc

N_PAD = 10240
ROWS = 2560
GRID = 4
H = 128
NG = 64
NC, NS = 2, 16
NW = NC * NS
BLK = 128
NBLK = 80
E_PAD = NW * NBLK * BLK
RPT = N_PAD // NS

_SC_MESH = plsc.VectorSubcoreMesh(
    core_axis_name="c", subcore_axis_name="s", num_cores=NC, num_subcores=NS)


def _fill_zero(zbuf):
    zero = jnp.zeros((16,), jnp.float32)

    def body(r, carry):
        for c in range(H // 16):
            zbuf[r, pl.ds(c * 16, 16)] = zero
        return carry

    lax.fori_loop(0, BLK, body, 0)


def _zero_slice(acc, zbuf, sid):
    _fill_zero(zbuf)
    for k in range(RPT // BLK):
        pltpu.sync_copy(zbuf, acc.at[pl.ds(sid * RPT + k * BLK, BLK)])


@functools.partial(
    pl.kernel,
    out_type=jax.ShapeDtypeStruct((NC * N_PAD, H), jnp.float32),
    mesh=_SC_MESH,
    scratch_types=[
        pltpu.VMEM((NBLK // 2, BLK), jnp.int32),
        pltpu.VMEM((BLK, H), jnp.float32),
        pltpu.VMEM((BLK, H), jnp.float32),
        pltpu.VMEM_SHARED((N_PAD, H), jnp.float32),
    ],
)
def _sc_degree(col_hbm, ones_hbm, out_hbm, colv, onesv, zbuf, acc):
    cid = lax.axis_index("c")
    sid = lax.axis_index("s")
    wid = cid * NS + sid
    pltpu.sync_copy(ones_hbm, onesv)
    _zero_slice(acc, zbuf, sid)
    plsc.subcore_barrier()

    def body(j, carry):
        pltpu.sync_copy(onesv, acc.at[colv.at[j]], add=True)
        return carry

    for ph in range(2):
        pltpu.sync_copy(col_hbm.at[2 * wid + ph], colv)
        lax.fori_loop(0, NBLK // 2, body, 0)
    plsc.subcore_barrier()
    pltpu.sync_copy(acc.at[pl.ds(sid * RPT, RPT)],
                    out_hbm.at[pl.ds(cid * N_PAD + sid * RPT, RPT)])


HALF = NBLK // 2


@functools.partial(
    pl.kernel,
    out_type=jax.ShapeDtypeStruct((NC * N_PAD, H), jnp.float32),
    mesh=_SC_MESH,
    scratch_types=[
        pltpu.VMEM((HALF, BLK), jnp.int32),
        pltpu.VMEM((HALF, BLK), jnp.int32),
        pltpu.VMEM((BLK, H), jnp.float32),
        pltpu.VMEM((BLK, H), jnp.float32),
        pltpu.VMEM_SHARED((N_PAD, H), jnp.float32),
        pltpu.SemaphoreType.DMA,
        pltpu.SemaphoreType.DMA,
    ],
)
def _sc_agg(rowi_hbm, coli_hbm, hp_hbm, out_hbm,
            rowv, colv, buf0, buf1, acc, sem0, sem1):
    cid = lax.axis_index("c")
    sid = lax.axis_index("s")
    wid = cid * NS + sid
    _zero_slice(acc, buf0, sid)
    plsc.subcore_barrier()

    bufs = (buf0, buf1)
    sems = (sem0, sem1)
    for ph in range(2):
        pltpu.sync_copy(rowi_hbm.at[2 * wid + ph], rowv)
        pltpu.sync_copy(coli_hbm.at[2 * wid + ph], colv)
        pltpu.async_copy(hp_hbm.at[rowv.at[0]], buf0, sem0)
        pltpu.async_copy(hp_hbm.at[rowv.at[1]], buf1, sem1)

        def body(t, carry):
            for b in range(2):
                j = 2 * t + b
                pltpu.make_async_copy(
                    hp_hbm.at[rowv.at[j]], bufs[b], sems[b]).wait()
                pltpu.sync_copy(bufs[b], acc.at[colv.at[j]], add=True)

                @pl.when(j + 2 < HALF)
                def _():
                    pltpu.async_copy(hp_hbm.at[rowv.at[j + 2]], bufs[b], sems[b])
            return carry

        lax.fori_loop(0, HALF // 2, body, 0)
    plsc.subcore_barrier()
    pltpu.sync_copy(acc.at[pl.ds(sid * RPT, RPT)],
                    out_hbm.at[pl.ds(cid * N_PAD + sid * RPT, RPT)])


def _dinv(cnt_ref):
    return lax.rsqrt(cnt_ref[0, :, 0:1] + cnt_ref[1, :, 0:1] + 1.0)


def _enc_body(x_ref, w1_ref, b1_ref, g_ref, be_ref, wc_ref, o_ref):
    h = jnp.maximum(
        jnp.dot(x_ref[...], w1_ref[...], preferred_element_type=jnp.float32)
        + b1_ref[...], 0.0)
    mu = jnp.mean(h, axis=-1, keepdims=True)
    var = jnp.mean((h - mu) * (h - mu), axis=-1, keepdims=True)
    h = (h - mu) * lax.rsqrt(var + 1e-5) * g_ref[...] + be_ref[...]
    o_ref[...] = jnp.dot(h, wc_ref[...], preferred_element_type=jnp.float32)


def _scale_body(cnt_ref, hw_ref, o_ref):
    o_ref[...] = hw_ref[...] * _dinv(cnt_ref)


def _combine_body(cnt_ref, p_ref, hp_ref, b_ref, w_ref, o_ref):
    dinv = _dinv(cnt_ref)
    h = jnp.maximum((p_ref[0] + p_ref[1] + hp_ref[...]) * dinv + b_ref[...], 0.0)
    o_ref[...] = jnp.dot(h, w_ref[...], preferred_element_type=jnp.float32) * dinv


def _final_body(cnt_ref, p_ref, hp_ref, b_ref, batch_ref,
                wm1_ref, bm1_ref, wm2_ref, bm2_ref, o_ref, sums, cnts):
    i = pl.program_id(0)

    @pl.when(i == 0)
    def _():
        sums[...] = jnp.zeros_like(sums)
        cnts[...] = jnp.zeros_like(cnts)

    dinv = _dinv(cnt_ref)
    h = jnp.maximum((p_ref[0] + p_ref[1] + hp_ref[...]) * dinv + b_ref[...], 0.0)
    b = batch_ref[0, 0, :]
    onehot = (b[:, None] == lax.broadcasted_iota(jnp.int32, (ROWS, NG), 1)
              ).astype(jnp.float32)
    sums[...] += lax.dot_general(onehot, h, (((0,), (0,)), ((), ())),
                                 preferred_element_type=jnp.float32)
    cnts[...] += jnp.sum(onehot, axis=0, keepdims=True)

    @pl.when(i == pl.num_programs(0) - 1)
    def _():
        means = sums[...] / jnp.maximum(cnts[...], 1.0).T
        z = jnp.maximum(
            jnp.dot(means, wm1_ref[...], preferred_element_type=jnp.float32)
            + bm1_ref[...], 0.0)
        o_ref[...] = (jnp.dot(z, wm2_ref[...], preferred_element_type=jnp.float32)
                      + bm2_ref[...])


def _full(shape):
    nd = len(shape)
    return pl.BlockSpec(shape, lambda i: (0,) * nd)


_ROWBLOCK = pl.BlockSpec((ROWS, H), lambda i: (i, 0))
_PBLOCK = pl.BlockSpec((NC, ROWS, H), lambda i: (0, i, 0))
_CNTBLOCK = pl.BlockSpec((NC, ROWS, 16), lambda i: (0, i, 0))

_tc_encoder = pl.pallas_call(
    _enc_body, grid=(GRID,),
    in_specs=[_ROWBLOCK, _full((H, H)), _full((1, H)), _full((1, H)),
              _full((1, H)), _full((H, H))],
    out_specs=_ROWBLOCK,
    out_shape=jax.ShapeDtypeStruct((N_PAD, H), jnp.float32))

_tc_scale = pl.pallas_call(
    _scale_body, grid=(GRID,),
    in_specs=[_CNTBLOCK, _ROWBLOCK],
    out_specs=_ROWBLOCK,
    out_shape=jax.ShapeDtypeStruct((N_PAD, H), jnp.float32))

_tc_combine = pl.pallas_call(
    _combine_body, grid=(GRID,),
    in_specs=[_CNTBLOCK, _PBLOCK, _ROWBLOCK, _full((1, H)), _full((H, H))],
    out_specs=_ROWBLOCK,
    out_shape=jax.ShapeDtypeStruct((N_PAD, H), jnp.float32))

_tc_final = pl.pallas_call(
    _final_body, grid=(GRID,),
    in_specs=[_CNTBLOCK, _PBLOCK, _ROWBLOCK, _full((1, H)),
              pl.BlockSpec((1, 1, ROWS), lambda i: (i, 0, 0)),
              _full((H, NG)), _full((1, NG)), _full((NG, 1)), _full((1, 1))],
    out_specs=_full((NG, 1)),
    out_shape=jax.ShapeDtypeStruct((NG, 1), jnp.float32),
    scratch_shapes=[pltpu.VMEM((NG, H), jnp.float32),
                    pltpu.VMEM((1, NG), jnp.float32)])


def kernel(x, edge_index, edge_attr, batch, W1, b1, gamma, beta, We, be,
           Wc0, bc0, Wc1, bc1, Wc2, bc2, Wm1, bm1, Wm2, bm2):
    n = x.shape[0]
    f32 = jnp.float32
    i32 = jnp.int32

    x_pad = jnp.zeros((N_PAD, H), f32).at[:n].set(x)
    epad = E_PAD - edge_index.shape[1]
    dummy = n + (jnp.arange(epad, dtype=i32) % (N_PAD - n))
    row3 = jnp.concatenate(
        [edge_index[0].astype(i32), dummy]).reshape(NW * 2, HALF, BLK)
    col3 = jnp.concatenate(
        [edge_index[1].astype(i32), dummy]).reshape(NW * 2, HALF, BLK)
    batch3 = jnp.concatenate(
        [batch.astype(i32), jnp.full((N_PAD - n,), NG, i32)]
    ).reshape(GRID, 1, ROWS)
    ones_blk = jnp.ones((BLK, H), f32)
    b1r, gr, ber = b1.reshape(1, H), gamma.reshape(1, H), beta.reshape(1, H)
    bc0r, bc1r, bc2r = bc0.reshape(1, H), bc1.reshape(1, H), bc2.reshape(1, H)
    bm1r, bm2r = bm1.reshape(1, NG), bm2.reshape(1, 1)

    cnt = _sc_degree(col3, ones_blk).reshape(
        NC, N_PAD, H)[:, :, :16]
    hw0 = _tc_encoder(x_pad, W1, b1r, gr, ber, Wc0)
    hp0 = _tc_scale(cnt, hw0)
    p1 = _sc_agg(row3, col3, hp0).reshape(NC, N_PAD, H)
    hp1 = _tc_combine(cnt, p1, hp0, bc0r, Wc1)
    p2 = _sc_agg(row3, col3, hp1).reshape(NC, N_PAD, H)
    hp2 = _tc_combine(cnt, p2, hp1, bc1r, Wc2)
    p3 = _sc_agg(row3, col3, hp2).reshape(NC, N_PAD, H)
    return _tc_final(cnt, p3, hp2, bc2r, batch3, Wm1, bm1r, Wm2, bm2r)


# speedup vs baseline: 26.5993x; 1.0437x over previous
import functools

import jax
import jax.numpy as jnp
from jax import lax
from jax.experimental import pallas as pl
from jax.experimental.pallas import tpu as pltpu
from jax.experimental.pallas import tpu_sc as plsc

N_PAD = 10240
ROWS = 2560
GRID = 4
H = 128
NG = 64
NC, NS = 2, 16
NW = NC * NS
BLK = 128
NBLK = 80
GB = 64
GNB = 160
GPH = 4
GQ = GNB // GPH
NBUF = 4
E_PAD = NW * NBLK * BLK
RPT = N_PAD // NS

_SC_MESH = plsc.VectorSubcoreMesh(
    core_axis_name="c", subcore_axis_name="s", num_cores=NC, num_subcores=NS)


def _fill_zero(zbuf):
    zero = jnp.zeros((16,), jnp.float32)

    def body(r, carry):
        for c in range(H // 16):
            zbuf[r, pl.ds(c * 16, 16)] = zero
        return carry

    lax.fori_loop(0, zbuf.shape[0], body, 0)


def _zero_slice(acc, zbuf, sid):
    _fill_zero(zbuf)
    r = zbuf.shape[0]
    for k in range(RPT // r):
        pltpu.sync_copy(zbuf, acc.at[pl.ds(sid * RPT + k * r, r)])


@functools.partial(
    pl.kernel,
    out_type=jax.ShapeDtypeStruct((NC * N_PAD, H), jnp.float32),
    mesh=_SC_MESH,
    scratch_types=[
        pltpu.VMEM((NBLK // 2, BLK), jnp.int32),
        pltpu.VMEM((BLK, H), jnp.float32),
        pltpu.VMEM((BLK, H), jnp.float32),
        pltpu.VMEM_SHARED((N_PAD, H), jnp.float32),
    ],
)
def _sc_degree(col_hbm, ones_hbm, out_hbm, colv, onesv, zbuf, acc):
    cid = lax.axis_index("c")
    sid = lax.axis_index("s")
    wid = cid * NS + sid
    pltpu.sync_copy(ones_hbm, onesv)
    _zero_slice(acc, zbuf, sid)
    plsc.subcore_barrier()

    def body(j, carry):
        pltpu.sync_copy(onesv, acc.at[colv.at[j]], add=True)
        return carry

    for ph in range(2):
        pltpu.sync_copy(col_hbm.at[2 * wid + ph], colv)
        lax.fori_loop(0, NBLK // 2, body, 0)
    plsc.subcore_barrier()
    pltpu.sync_copy(acc.at[pl.ds(sid * RPT, RPT)],
                    out_hbm.at[pl.ds(cid * N_PAD + sid * RPT, RPT)])


@functools.partial(
    pl.kernel,
    out_type=jax.ShapeDtypeStruct((NC * N_PAD, H), jnp.float32),
    mesh=_SC_MESH,
    scratch_types=[
        pltpu.VMEM((GQ, GB), jnp.int32),
        pltpu.VMEM((GQ, GB), jnp.int32),
        [pltpu.VMEM((GB, H), jnp.float32)] * NBUF,
        [pltpu.SemaphoreType.DMA] * NBUF,
        pltpu.VMEM_SHARED((N_PAD, H), jnp.float32),
    ],
)
def _sc_agg(rowi_hbm, coli_hbm, hp_hbm, out_hbm,
            rowv, colv, bufs, sems, acc):
    cid = lax.axis_index("c")
    sid = lax.axis_index("s")
    wid = cid * NS + sid
    _zero_slice(acc, bufs[0], sid)
    plsc.subcore_barrier()

    for ph in range(GPH):
        pltpu.sync_copy(rowi_hbm.at[GPH * wid + ph], rowv)
        pltpu.sync_copy(coli_hbm.at[GPH * wid + ph], colv)
        for b in range(NBUF):
            pltpu.async_copy(hp_hbm.at[rowv.at[b]], bufs[b], sems[b])

        def body(t, carry):
            for b in range(NBUF):
                j = NBUF * t + b
                pltpu.make_async_copy(
                    hp_hbm.at[rowv.at[j]], bufs[b], sems[b]).wait()
                pltpu.sync_copy(bufs[b], acc.at[colv.at[j]], add=True)

                @pl.when(j + NBUF < GQ)
                def _():
                    pltpu.async_copy(
                        hp_hbm.at[rowv.at[j + NBUF]], bufs[b], sems[b])
            return carry

        lax.fori_loop(0, GQ // NBUF, body, 0)
    plsc.subcore_barrier()
    pltpu.sync_copy(acc.at[pl.ds(sid * RPT, RPT)],
                    out_hbm.at[pl.ds(cid * N_PAD + sid * RPT, RPT)])


def _dinv(cnt_ref):
    return lax.rsqrt(cnt_ref[0, :, 0:1] + cnt_ref[1, :, 0:1] + 1.0)


def _enc_body(x_ref, w1_ref, b1_ref, g_ref, be_ref, wc_ref, o_ref):
    h = jnp.maximum(
        jnp.dot(x_ref[...], w1_ref[...], preferred_element_type=jnp.float32)
        + b1_ref[...], 0.0)
    mu = jnp.mean(h, axis=-1, keepdims=True)
    var = jnp.mean((h - mu) * (h - mu), axis=-1, keepdims=True)
    h = (h - mu) * lax.rsqrt(var + 1e-5) * g_ref[...] + be_ref[...]
    o_ref[...] = jnp.dot(h, wc_ref[...], preferred_element_type=jnp.float32)


def _scale_body(cnt_ref, hw_ref, o_ref):
    o_ref[...] = hw_ref[...] * _dinv(cnt_ref)


def _combine_body(cnt_ref, p_ref, hp_ref, b_ref, w_ref, o_ref):
    dinv = _dinv(cnt_ref)
    h = jnp.maximum((p_ref[0] + p_ref[1] + hp_ref[...]) * dinv + b_ref[...], 0.0)
    o_ref[...] = jnp.dot(h, w_ref[...], preferred_element_type=jnp.float32) * dinv


def _final_body(cnt_ref, p_ref, hp_ref, b_ref, batch_ref,
                wm1_ref, bm1_ref, wm2_ref, bm2_ref, o_ref, sums, cnts):
    i = pl.program_id(0)

    @pl.when(i == 0)
    def _():
        sums[...] = jnp.zeros_like(sums)
        cnts[...] = jnp.zeros_like(cnts)

    dinv = _dinv(cnt_ref)
    h = jnp.maximum((p_ref[0] + p_ref[1] + hp_ref[...]) * dinv + b_ref[...], 0.0)
    b = batch_ref[0, 0, :]
    onehot = (b[:, None] == lax.broadcasted_iota(jnp.int32, (ROWS, NG), 1)
              ).astype(jnp.float32)
    sums[...] += lax.dot_general(onehot, h, (((0,), (0,)), ((), ())),
                                 preferred_element_type=jnp.float32)
    cnts[...] += jnp.sum(onehot, axis=0, keepdims=True)

    @pl.when(i == pl.num_programs(0) - 1)
    def _():
        means = sums[...] / jnp.maximum(cnts[...], 1.0).T
        z = jnp.maximum(
            jnp.dot(means, wm1_ref[...], preferred_element_type=jnp.float32)
            + bm1_ref[...], 0.0)
        o_ref[...] = (jnp.dot(z, wm2_ref[...], preferred_element_type=jnp.float32)
                      + bm2_ref[...])


def _full(shape):
    nd = len(shape)
    return pl.BlockSpec(shape, lambda i: (0,) * nd)


_ROWBLOCK = pl.BlockSpec((ROWS, H), lambda i: (i, 0))
_PBLOCK = pl.BlockSpec((NC, ROWS, H), lambda i: (0, i, 0))
_CNTBLOCK = pl.BlockSpec((NC, ROWS, 16), lambda i: (0, i, 0))

_tc_encoder = pl.pallas_call(
    _enc_body, grid=(GRID,),
    in_specs=[_ROWBLOCK, _full((H, H)), _full((1, H)), _full((1, H)),
              _full((1, H)), _full((H, H))],
    out_specs=_ROWBLOCK,
    out_shape=jax.ShapeDtypeStruct((N_PAD, H), jnp.float32))

_tc_scale = pl.pallas_call(
    _scale_body, grid=(GRID,),
    in_specs=[_CNTBLOCK, _ROWBLOCK],
    out_specs=_ROWBLOCK,
    out_shape=jax.ShapeDtypeStruct((N_PAD, H), jnp.float32))

_tc_combine = pl.pallas_call(
    _combine_body, grid=(GRID,),
    in_specs=[_CNTBLOCK, _PBLOCK, _ROWBLOCK, _full((1, H)), _full((H, H))],
    out_specs=_ROWBLOCK,
    out_shape=jax.ShapeDtypeStruct((N_PAD, H), jnp.float32))

_tc_final = pl.pallas_call(
    _final_body, grid=(GRID,),
    in_specs=[_CNTBLOCK, _PBLOCK, _ROWBLOCK, _full((1, H)),
              pl.BlockSpec((1, 1, ROWS), lambda i: (i, 0, 0)),
              _full((H, NG)), _full((1, NG)), _full((NG, 1)), _full((1, 1))],
    out_specs=_full((NG, 1)),
    out_shape=jax.ShapeDtypeStruct((NG, 1), jnp.float32),
    scratch_shapes=[pltpu.VMEM((NG, H), jnp.float32),
                    pltpu.VMEM((1, NG), jnp.float32)])


def kernel(x, edge_index, edge_attr, batch, W1, b1, gamma, beta, We, be,
           Wc0, bc0, Wc1, bc1, Wc2, bc2, Wm1, bm1, Wm2, bm2):
    n = x.shape[0]
    f32 = jnp.float32
    i32 = jnp.int32

    x_pad = jnp.zeros((N_PAD, H), f32).at[:n].set(x)
    epad = E_PAD - edge_index.shape[1]
    dummy = n + (jnp.arange(epad, dtype=i32) % (N_PAD - n))
    row_flat = jnp.concatenate([edge_index[0].astype(i32), dummy])
    col_flat = jnp.concatenate([edge_index[1].astype(i32), dummy])
    row3 = row_flat.reshape(NW * GPH, GQ, GB)
    col3 = col_flat.reshape(NW * GPH, GQ, GB)
    col3d = col_flat.reshape(NW * 2, NBLK // 2, BLK)
    batch3 = jnp.concatenate(
        [batch.astype(i32), jnp.full((N_PAD - n,), NG, i32)]
    ).reshape(GRID, 1, ROWS)
    ones_blk = jnp.ones((BLK, H), f32)
    b1r, gr, ber = b1.reshape(1, H), gamma.reshape(1, H), beta.reshape(1, H)
    bc0r, bc1r, bc2r = bc0.reshape(1, H), bc1.reshape(1, H), bc2.reshape(1, H)
    bm1r, bm2r = bm1.reshape(1, NG), bm2.reshape(1, 1)

    cnt = _sc_degree(col3d, ones_blk).reshape(
        NC, N_PAD, H)[:, :, :16]
    hw0 = _tc_encoder(x_pad, W1, b1r, gr, ber, Wc0)
    hp0 = _tc_scale(cnt, hw0)
    p1 = _sc_agg(row3, col3, hp0).reshape(NC, N_PAD, H)
    hp1 = _tc_combine(cnt, p1, hp0, bc0r, Wc1)
    p2 = _sc_agg(row3, col3, hp1).reshape(NC, N_PAD, H)
    hp2 = _tc_combine(cnt, p2, hp1, bc1r, Wc2)
    p3 = _sc_agg(row3, col3, hp2).reshape(NC, N_PAD, H)
    return _tc_final(cnt, p3, hp2, bc2r, batch3, Wm1, bm1r, Wm2, bm2r)
